# baseline (speedup 1.0000x reference)
import functools

import jax
import jax.numpy as jnp
from jax import lax
from jax.experimental import pallas as pl
from jax.experimental.pallas import tpu as pltpu
from jax.experimental.pallas import tpu_sc as plsc

N_NODES = 10000
N_EDGES = 320000
D = 128

NC, NS = 2, 16
NW = NC * NS
BLK = 128
NBLK = 80
SUPER = 16
NSUP = NBLK // SUPER
TOT_SUP = NW * NSUP
FAST_C = 0
FS = 10
SS = 0
EPW = NBLK * BLK
E_PAD = NW * EPW
N_PAD = 10240
DEG_PW = N_PAD // NS
AGG_PW = N_PAD // NS

_mesh = plsc.VectorSubcoreMesh(core_axis_name="c", subcore_axis_name="s")


def _deg_body(colp, ewp, degp, degsp, col_v, ew_v, zbuf, dbuf):
    c = lax.axis_index("c")
    s = lax.axis_index("s")
    w = c * NS + s

    def zrow(i, _):
        zbuf[pl.ds(i * 16, 16)] = jnp.zeros((16,), jnp.float32)
        return 0

    lax.fori_loop(0, DEG_PW // 16, zrow, 0)
    pltpu.sync_copy(zbuf, degsp.at[pl.ds(s * DEG_PW, DEG_PW)])
    pltpu.sync_copy(colp.at[w], col_v)
    pltpu.sync_copy(ewp.at[w], ew_v)
    plsc.subcore_barrier()

    def blk(g, _):
        pltpu.sync_copy(ew_v.at[g], degsp.at[col_v.at[g]], add=True)
        return 0

    lax.fori_loop(0, NBLK, blk, 0)
    plsc.subcore_barrier()
    pltpu.sync_copy(degsp.at[pl.ds(s * DEG_PW, DEG_PW)], dbuf)
    pltpu.sync_copy(dbuf, degp.at[c, pl.ds(s * DEG_PW, DEG_PW)])


@functools.partial(
    pl.kernel,
    out_type=jax.ShapeDtypeStruct((NC, N_PAD), jnp.float32),
    mesh=_mesh,
    scratch_types=[
        pltpu.VMEM_SHARED((N_PAD,), jnp.float32),
    ],
)
def _sc_deg(colp, ewp, degp, degsp):
    pl.run_scoped(
        lambda col_v, ew_v, zbuf, dbuf: _deg_body(
            colp, ewp, degp, degsp, col_v, ew_v, zbuf, dbuf),
        pltpu.VMEM((NBLK, BLK), jnp.int32),
        pltpu.VMEM((NBLK, BLK), jnp.float32),
        pltpu.VMEM((DEG_PW,), jnp.float32),
        pltpu.VMEM((DEG_PW,), jnp.float32),
    )


def _scale_block(buf, ew_v, g):
    def grp(t, _):
        wv = ew_v[g, pl.ds(t * 16, 16)]
        for lane in range(16):
            wgt = wv[lane]
            e = t * 16 + lane
            for j in range(D // 16):
                sl = pl.ds(j * 16, 16)
                buf[e, sl] = buf[e, sl] * wgt
        return 0

    lax.fori_loop(0, BLK // 16, grp, 0)


def _agg_body(y, rowp, colp, ewp, aggp, aggsp, sem0, sem1, sem0b, sem1b,
              ssem0, ssem1, row_v, col_v, ew_v, buf0, buf1):
    c = lax.axis_index("c")
    s = lax.axis_index("s")

    def zrow(i, _):
        for j in range(D // 16):
            buf0[i, pl.ds(j * 16, 16)] = jnp.zeros((16,), jnp.float32)
        return 0

    lax.fori_loop(0, BLK, zrow, 0)
    for k in range(AGG_PW // BLK):
        pltpu.sync_copy(buf0, aggsp.at[pl.ds(s * AGG_PW + k * BLK, BLK)])
    plsc.subcore_barrier()

    is_fast = c == FAST_C
    cnt = jnp.where(is_fast, FS, SS)
    base = jnp.where(is_fast, s * FS, NS * FS + s * SS)

    def sup(u, _):
        su = base + u
        pltpu.sync_copy(rowp.at[su], row_v)
        pltpu.sync_copy(colp.at[su], col_v)
        pltpu.sync_copy(ewp.at[su], ew_v)
        def gissue(g, buf, sa, sb):
            pltpu.async_copy(y.at[row_v.at[g, pl.ds(0, 64)]],
                             buf.at[pl.ds(0, 64)], sa)
            pltpu.async_copy(y.at[row_v.at[g, pl.ds(64, 64)]],
                             buf.at[pl.ds(64, 64)], sb)

        def gwait(g, buf, sa, sb):
            pltpu.make_async_copy(y.at[row_v.at[g, pl.ds(0, 64)]],
                                  buf.at[pl.ds(0, 64)], sa).wait()
            pltpu.make_async_copy(y.at[row_v.at[g, pl.ds(64, 64)]],
                                  buf.at[pl.ds(64, 64)], sb).wait()

        gissue(0, buf0, sem0, sem0b)

        def step(i, _):
            g0 = 2 * i
            g1 = 2 * i + 1

            @pl.when(i > 0)
            def _():
                pltpu.make_async_copy(
                    buf1, aggsp.at[col_v.at[g0 - 1]], ssem1).wait()

            gissue(g1, buf1, sem1, sem1b)
            gwait(g0, buf0, sem0, sem0b)
            _scale_block(buf0, ew_v, g0)
            pltpu.async_copy(buf0, aggsp.at[col_v.at[g0]], ssem0, add=True)

            gwait(g1, buf1, sem1, sem1b)
            _scale_block(buf1, ew_v, g1)
            pltpu.make_async_copy(
                buf0, aggsp.at[col_v.at[g0]], ssem0).wait()

            @pl.when(g1 + 1 < SUPER)
            def _():
                gissue(g1 + 1, buf0, sem0, sem0b)

            pltpu.async_copy(buf1, aggsp.at[col_v.at[g1]], ssem1, add=True)
            return 0

        lax.fori_loop(0, SUPER // 2, step, 0)
        pltpu.make_async_copy(
            buf1, aggsp.at[col_v.at[SUPER - 1]], ssem1).wait()
        return 0

    lax.fori_loop(0, cnt, sup, 0)
    plsc.subcore_barrier()

    for k in range(AGG_PW // BLK):
        r0 = s * AGG_PW + k * BLK
        pltpu.sync_copy(aggsp.at[pl.ds(r0, BLK)], buf0)
        pltpu.sync_copy(buf0, aggp.at[c, pl.ds(r0, BLK)])


@functools.partial(
    pl.kernel,
    out_type=jax.ShapeDtypeStruct((NC, N_PAD, D), jnp.float32),
    mesh=_mesh,
    scratch_types=[
        pltpu.VMEM_SHARED((N_PAD, D), jnp.float32),
        pltpu.SemaphoreType.DMA,
        pltpu.SemaphoreType.DMA,
        pltpu.SemaphoreType.DMA,
        pltpu.SemaphoreType.DMA,
        pltpu.SemaphoreType.DMA,
        pltpu.SemaphoreType.DMA,
    ],
)
def _sc_agg(y, rowp, colp, ewp, aggp, aggsp, sem0, sem1, sem0b, sem1b,
            ssem0, ssem1):
    pl.run_scoped(
        lambda row_v, col_v, ew_v, buf0, buf1: _agg_body(
            y, rowp, colp, ewp, aggp, aggsp, sem0, sem1, sem0b, sem1b,
            ssem0, ssem1, row_v, col_v, ew_v, buf0, buf1),
        pltpu.VMEM((SUPER, BLK), jnp.int32),
        pltpu.VMEM((SUPER, BLK), jnp.int32),
        pltpu.VMEM((SUPER, BLK), jnp.float32),
        pltpu.VMEM((BLK, D), jnp.float32),
        pltpu.VMEM((BLK, D), jnp.float32),
    )


_EWB = 12800


def _ew_body(eaT_ref, ew_ref):
    ew_ref[...] = 0.5 * (eaT_ref[0:1, :] + eaT_ref[1:2, :])


_tc_ew = pl.pallas_call(
    _ew_body,
    grid=(N_EDGES // _EWB,),
    in_specs=[pl.BlockSpec((4, _EWB), lambda i: (0, i))],
    out_specs=pl.BlockSpec((1, _EWB), lambda i: (0, i)),
    out_shape=jax.ShapeDtypeStruct((1, N_EDGES), jnp.float32),
)

_NB = 1000


def _l1_body(x_ref, w1_ref, degT_ref, xw_ref, y_ref, dinv_ref):
    xw = jnp.dot(x_ref[...], w1_ref[...], preferred_element_type=jnp.float32)
    deg = degT_ref[:, 0:1] + degT_ref[:, 1:2] + 1.0
    dinv = lax.rsqrt(deg)
    xw_ref[...] = xw
    y_ref[...] = dinv * xw
    dinv_ref[...] = dinv


_tc_l1 = pl.pallas_call(
    _l1_body,
    grid=(N_NODES // _NB,),
    in_specs=[
        pl.BlockSpec((_NB, D), lambda i: (i, 0)),
        pl.BlockSpec((D, D), lambda i: (0, 0)),
        pl.BlockSpec((_NB, 2), lambda i: (i, 0)),
    ],
    out_specs=[
        pl.BlockSpec((_NB, D), lambda i: (i, 0)),
        pl.BlockSpec((_NB, D), lambda i: (i, 0)),
        pl.BlockSpec((_NB, 1), lambda i: (i, 0)),
    ],
    out_shape=[
        jax.ShapeDtypeStruct((N_NODES, D), jnp.float32),
        jax.ShapeDtypeStruct((N_NODES, D), jnp.float32),
        jax.ShapeDtypeStruct((N_NODES, 1), jnp.float32),
    ],
)


def _l2_body(a0_ref, a1_ref, xw1_ref, dinv_ref, b1_ref, w2_ref,
             xw2_ref, y2_ref):
    dinv = dinv_ref[...]
    h = dinv * (a0_ref[...] + a1_ref[...]) + dinv * dinv * xw1_ref[...]
    h = jnp.maximum(h + b1_ref[...], 0.0)
    xw2 = jnp.dot(h, w2_ref[...], preferred_element_type=jnp.float32)
    xw2_ref[...] = xw2
    y2_ref[...] = dinv * xw2


_tc_l2 = pl.pallas_call(
    _l2_body,
    grid=(N_NODES // _NB,),
    in_specs=[
        pl.BlockSpec((_NB, D), lambda i: (i, 0)),
        pl.BlockSpec((_NB, D), lambda i: (i, 0)),
        pl.BlockSpec((_NB, D), lambda i: (i, 0)),
        pl.BlockSpec((_NB, 1), lambda i: (i, 0)),
        pl.BlockSpec((1, D), lambda i: (0, 0)),
        pl.BlockSpec((D, D), lambda i: (0, 0)),
    ],
    out_specs=[
        pl.BlockSpec((_NB, D), lambda i: (i, 0)),
        pl.BlockSpec((_NB, D), lambda i: (i, 0)),
    ],
    out_shape=[
        jax.ShapeDtypeStruct((N_NODES, D), jnp.float32),
        jax.ShapeDtypeStruct((N_NODES, D), jnp.float32),
    ],
)


def _l3_body(a0_ref, a1_ref, xw2_ref, dinv_ref, b2_ref, wc_ref, bc_ref,
             out_ref):
    dinv = dinv_ref[...]
    h = dinv * (a0_ref[...] + a1_ref[...]) + dinv * dinv * xw2_ref[...]
    h = jnp.maximum(h + b2_ref[...], 0.0)
    out_ref[...] = jnp.dot(h, wc_ref[...],
                           preferred_element_type=jnp.float32) + bc_ref[...]


_tc_l3 = pl.pallas_call(
    _l3_body,
    grid=(N_NODES // _NB,),
    in_specs=[
        pl.BlockSpec((_NB, D), lambda i: (i, 0)),
        pl.BlockSpec((_NB, D), lambda i: (i, 0)),
        pl.BlockSpec((_NB, D), lambda i: (i, 0)),
        pl.BlockSpec((_NB, 1), lambda i: (i, 0)),
        pl.BlockSpec((1, D), lambda i: (0, 0)),
        pl.BlockSpec((D, 2), lambda i: (0, 0)),
        pl.BlockSpec((1, 2), lambda i: (0, 0)),
    ],
    out_specs=pl.BlockSpec((_NB, 2), lambda i: (i, 0)),
    out_shape=jax.ShapeDtypeStruct((N_NODES, 2), jnp.float32),
)


def kernel(x, edge_index, edge_attr, W1, b1, W2, b2, Wc, bc):
    ei = edge_index.astype(jnp.int32)
    pad = E_PAD - N_EDGES
    rowp = jnp.concatenate([ei[0], jnp.zeros((pad,), jnp.int32)])
    colp = jnp.concatenate([ei[1], jnp.zeros((pad,), jnp.int32)])
    rowp4 = rowp.reshape(TOT_SUP, SUPER, BLK)
    colp4 = colp.reshape(TOT_SUP, SUPER, BLK)
    colp = colp.reshape(NW, NBLK, BLK)

    ew = _tc_ew(edge_attr.T.astype(jnp.float32)).reshape(N_EDGES)
    ewp = jnp.concatenate([ew, jnp.zeros((pad,), jnp.float32)])
    ewp4 = ewp.reshape(TOT_SUP, SUPER, BLK)
    ewp = ewp.reshape(NW, NBLK, BLK)

    degp = _sc_deg(colp, ewp)
    degT = degp.T[:N_NODES]

    xw1, y1, dinv = _tc_l1(x, W1, degT)
    agg1 = _sc_agg(y1, rowp4, colp4, ewp4)
    xw2, y2 = _tc_l2(agg1[0], agg1[1], xw1, dinv, b1.reshape(1, D), W2)
    agg2 = _sc_agg(y2, rowp4, colp4, ewp4)
    return _tc_l3(agg2[0], agg2[1], xw2, dinv, b2.reshape(1, D), Wc,
                  bc.reshape(1, 2))


# speedup vs baseline: 1.6014x; 1.6014x over previous
import functools

import jax
import jax.numpy as jnp
from jax import lax
from jax.experimental import pallas as pl
from jax.experimental.pallas import tpu as pltpu
from jax.experimental.pallas import tpu_sc as plsc

N_NODES = 10000
N_EDGES = 320000
D = 128

NC, NS = 2, 16
NW = NC * NS
BLK = 128
NBLK = 80
SUPER = 16
NSUP = NBLK // SUPER
TOT_SUP = NW * NSUP
FAST_C = 0
FS = 9
SS = 1
EPW = NBLK * BLK
E_PAD = NW * EPW
N_PAD = 10240
DEG_PW = N_PAD // NS
AGG_PW = N_PAD // NS

_mesh = plsc.VectorSubcoreMesh(core_axis_name="c", subcore_axis_name="s")


def _deg_body(colp, ewp, degp, degsp, col_v, ew_v, zbuf, dbuf):
    c = lax.axis_index("c")
    s = lax.axis_index("s")
    w = c * NS + s

    def zrow(i, _):
        zbuf[pl.ds(i * 16, 16)] = jnp.zeros((16,), jnp.float32)
        return 0

    lax.fori_loop(0, DEG_PW // 16, zrow, 0)
    pltpu.sync_copy(zbuf, degsp.at[pl.ds(s * DEG_PW, DEG_PW)])
    pltpu.sync_copy(colp.at[w], col_v)
    pltpu.sync_copy(ewp.at[w], ew_v)
    plsc.subcore_barrier()

    def blk(g, _):
        pltpu.sync_copy(ew_v.at[g], degsp.at[col_v.at[g]], add=True)
        return 0

    lax.fori_loop(0, NBLK, blk, 0)
    plsc.subcore_barrier()
    pltpu.sync_copy(degsp.at[pl.ds(s * DEG_PW, DEG_PW)], dbuf)
    pltpu.sync_copy(dbuf, degp.at[c, pl.ds(s * DEG_PW, DEG_PW)])


@functools.partial(
    pl.kernel,
    out_type=jax.ShapeDtypeStruct((NC, N_PAD), jnp.float32),
    mesh=_mesh,
    scratch_types=[
        pltpu.VMEM_SHARED((N_PAD,), jnp.float32),
    ],
)
def _sc_deg(colp, ewp, degp, degsp):
    pl.run_scoped(
        lambda col_v, ew_v, zbuf, dbuf: _deg_body(
            colp, ewp, degp, degsp, col_v, ew_v, zbuf, dbuf),
        pltpu.VMEM((NBLK, BLK), jnp.int32),
        pltpu.VMEM((NBLK, BLK), jnp.float32),
        pltpu.VMEM((DEG_PW,), jnp.float32),
        pltpu.VMEM((DEG_PW,), jnp.float32),
    )


def _scale_block(buf, ew_v, g):
    def grp(t, _):
        wv = ew_v[g, pl.ds(t * 16, 16)]
        for lane in range(16):
            wgt = wv[lane]
            e = t * 16 + lane
            for j in range(D // 16):
                sl = pl.ds(j * 16, 16)
                buf[e, sl] = buf[e, sl] * wgt
        return 0

    lax.fori_loop(0, BLK // 16, grp, 0)


def _agg_body(y, rowp, colp, ewp, aggp, aggsp, sem0, sem1, sem0b, sem1b,
              ssem0, ssem1, row_v, col_v, ew_v, buf0, buf1):
    c = lax.axis_index("c")
    s = lax.axis_index("s")

    def zrow(i, _):
        for j in range(D // 16):
            buf0[i, pl.ds(j * 16, 16)] = jnp.zeros((16,), jnp.float32)
        return 0

    lax.fori_loop(0, BLK, zrow, 0)
    for k in range(AGG_PW // BLK):
        pltpu.sync_copy(buf0, aggsp.at[pl.ds(s * AGG_PW + k * BLK, BLK)])
    plsc.subcore_barrier()

    is_fast = c == FAST_C
    cnt = jnp.where(is_fast, FS, SS)
    base = jnp.where(is_fast, s * FS, NS * FS + s * SS)

    def sup(u, _):
        su = base + u
        pltpu.sync_copy(rowp.at[su], row_v)
        pltpu.sync_copy(colp.at[su], col_v)
        pltpu.sync_copy(ewp.at[su], ew_v)
        def gissue(g, buf, sa, sb):
            pltpu.async_copy(y.at[row_v.at[g, pl.ds(0, 64)]],
                             buf.at[pl.ds(0, 64)], sa)
            pltpu.async_copy(y.at[row_v.at[g, pl.ds(64, 64)]],
                             buf.at[pl.ds(64, 64)], sb)

        def gwait(g, buf, sa, sb):
            pltpu.make_async_copy(y.at[row_v.at[g, pl.ds(0, 64)]],
                                  buf.at[pl.ds(0, 64)], sa).wait()
            pltpu.make_async_copy(y.at[row_v.at[g, pl.ds(64, 64)]],
                                  buf.at[pl.ds(64, 64)], sb).wait()

        gissue(0, buf0, sem0, sem0b)

        def step(i, _):
            g0 = 2 * i
            g1 = 2 * i + 1

            @pl.when(i > 0)
            def _():
                pltpu.make_async_copy(
                    buf1, aggsp.at[col_v.at[g0 - 1]], ssem1).wait()

            gissue(g1, buf1, sem1, sem1b)
            gwait(g0, buf0, sem0, sem0b)
            _scale_block(buf0, ew_v, g0)
            pltpu.async_copy(buf0, aggsp.at[col_v.at[g0]], ssem0, add=True)

            gwait(g1, buf1, sem1, sem1b)
            _scale_block(buf1, ew_v, g1)
            pltpu.make_async_copy(
                buf0, aggsp.at[col_v.at[g0]], ssem0).wait()

            @pl.when(g1 + 1 < SUPER)
            def _():
                gissue(g1 + 1, buf0, sem0, sem0b)

            pltpu.async_copy(buf1, aggsp.at[col_v.at[g1]], ssem1, add=True)
            return 0

        lax.fori_loop(0, SUPER // 2, step, 0)
        pltpu.make_async_copy(
            buf1, aggsp.at[col_v.at[SUPER - 1]], ssem1).wait()
        return 0

    lax.fori_loop(0, cnt, sup, 0)
    plsc.subcore_barrier()

    for k in range(AGG_PW // BLK):
        r0 = s * AGG_PW + k * BLK
        pltpu.sync_copy(aggsp.at[pl.ds(r0, BLK)], buf0)
        pltpu.sync_copy(buf0, aggp.at[c, pl.ds(r0, BLK)])


@functools.partial(
    pl.kernel,
    out_type=jax.ShapeDtypeStruct((NC, N_PAD, D), jnp.float32),
    mesh=_mesh,
    scratch_types=[
        pltpu.VMEM_SHARED((N_PAD, D), jnp.float32),
        pltpu.SemaphoreType.DMA,
        pltpu.SemaphoreType.DMA,
        pltpu.SemaphoreType.DMA,
        pltpu.SemaphoreType.DMA,
        pltpu.SemaphoreType.DMA,
        pltpu.SemaphoreType.DMA,
    ],
)
def _sc_agg(y, rowp, colp, ewp, aggp, aggsp, sem0, sem1, sem0b, sem1b,
            ssem0, ssem1):
    pl.run_scoped(
        lambda row_v, col_v, ew_v, buf0, buf1: _agg_body(
            y, rowp, colp, ewp, aggp, aggsp, sem0, sem1, sem0b, sem1b,
            ssem0, ssem1, row_v, col_v, ew_v, buf0, buf1),
        pltpu.VMEM((SUPER, BLK), jnp.int32),
        pltpu.VMEM((SUPER, BLK), jnp.int32),
        pltpu.VMEM((SUPER, BLK), jnp.float32),
        pltpu.VMEM((BLK, D), jnp.float32),
        pltpu.VMEM((BLK, D), jnp.float32),
    )


_EWB = 12800


def _ew_body(eaT_ref, ew_ref):
    ew_ref[...] = 0.5 * (eaT_ref[0:1, :] + eaT_ref[1:2, :])


_tc_ew = pl.pallas_call(
    _ew_body,
    grid=(N_EDGES // _EWB,),
    in_specs=[pl.BlockSpec((4, _EWB), lambda i: (0, i))],
    out_specs=pl.BlockSpec((1, _EWB), lambda i: (0, i)),
    out_shape=jax.ShapeDtypeStruct((1, N_EDGES), jnp.float32),
)

_NB = 1000


def _l1_body(x_ref, w1_ref, degT_ref, xw_ref, y_ref, dinv_ref):
    xw = jnp.dot(x_ref[...], w1_ref[...], preferred_element_type=jnp.float32)
    deg = degT_ref[:, 0:1] + degT_ref[:, 1:2] + 1.0
    dinv = lax.rsqrt(deg)
    xw_ref[...] = xw
    y_ref[...] = dinv * xw
    dinv_ref[...] = dinv


_tc_l1 = pl.pallas_call(
    _l1_body,
    grid=(N_NODES // _NB,),
    in_specs=[
        pl.BlockSpec((_NB, D), lambda i: (i, 0)),
        pl.BlockSpec((D, D), lambda i: (0, 0)),
        pl.BlockSpec((_NB, 2), lambda i: (i, 0)),
    ],
    out_specs=[
        pl.BlockSpec((_NB, D), lambda i: (i, 0)),
        pl.BlockSpec((_NB, D), lambda i: (i, 0)),
        pl.BlockSpec((_NB, 1), lambda i: (i, 0)),
    ],
    out_shape=[
        jax.ShapeDtypeStruct((N_NODES, D), jnp.float32),
        jax.ShapeDtypeStruct((N_NODES, D), jnp.float32),
        jax.ShapeDtypeStruct((N_NODES, 1), jnp.float32),
    ],
)


def _l2_body(a0_ref, a1_ref, xw1_ref, dinv_ref, b1_ref, w2_ref,
             xw2_ref, y2_ref):
    dinv = dinv_ref[...]
    h = dinv * (a0_ref[...] + a1_ref[...]) + dinv * dinv * xw1_ref[...]
    h = jnp.maximum(h + b1_ref[...], 0.0)
    xw2 = jnp.dot(h, w2_ref[...], preferred_element_type=jnp.float32)
    xw2_ref[...] = xw2
    y2_ref[...] = dinv * xw2


_tc_l2 = pl.pallas_call(
    _l2_body,
    grid=(N_NODES // _NB,),
    in_specs=[
        pl.BlockSpec((_NB, D), lambda i: (i, 0)),
        pl.BlockSpec((_NB, D), lambda i: (i, 0)),
        pl.BlockSpec((_NB, D), lambda i: (i, 0)),
        pl.BlockSpec((_NB, 1), lambda i: (i, 0)),
        pl.BlockSpec((1, D), lambda i: (0, 0)),
        pl.BlockSpec((D, D), lambda i: (0, 0)),
    ],
    out_specs=[
        pl.BlockSpec((_NB, D), lambda i: (i, 0)),
        pl.BlockSpec((_NB, D), lambda i: (i, 0)),
    ],
    out_shape=[
        jax.ShapeDtypeStruct((N_NODES, D), jnp.float32),
        jax.ShapeDtypeStruct((N_NODES, D), jnp.float32),
    ],
)


def _l3_body(a0_ref, a1_ref, xw2_ref, dinv_ref, b2_ref, wc_ref, bc_ref,
             out_ref):
    dinv = dinv_ref[...]
    h = dinv * (a0_ref[...] + a1_ref[...]) + dinv * dinv * xw2_ref[...]
    h = jnp.maximum(h + b2_ref[...], 0.0)
    out_ref[...] = jnp.dot(h, wc_ref[...],
                           preferred_element_type=jnp.float32) + bc_ref[...]


_tc_l3 = pl.pallas_call(
    _l3_body,
    grid=(N_NODES // _NB,),
    in_specs=[
        pl.BlockSpec((_NB, D), lambda i: (i, 0)),
        pl.BlockSpec((_NB, D), lambda i: (i, 0)),
        pl.BlockSpec((_NB, D), lambda i: (i, 0)),
        pl.BlockSpec((_NB, 1), lambda i: (i, 0)),
        pl.BlockSpec((1, D), lambda i: (0, 0)),
        pl.BlockSpec((D, 2), lambda i: (0, 0)),
        pl.BlockSpec((1, 2), lambda i: (0, 0)),
    ],
    out_specs=pl.BlockSpec((_NB, 2), lambda i: (i, 0)),
    out_shape=jax.ShapeDtypeStruct((N_NODES, 2), jnp.float32),
)


def kernel(x, edge_index, edge_attr, W1, b1, W2, b2, Wc, bc):
    ei = edge_index.astype(jnp.int32)
    pad = E_PAD - N_EDGES
    rowp = jnp.concatenate([ei[0], jnp.zeros((pad,), jnp.int32)])
    colp = jnp.concatenate([ei[1], jnp.zeros((pad,), jnp.int32)])
    rowp4 = rowp.reshape(TOT_SUP, SUPER, BLK)
    colp4 = colp.reshape(TOT_SUP, SUPER, BLK)
    colp = colp.reshape(NW, NBLK, BLK)

    ew = _tc_ew(edge_attr.T.astype(jnp.float32)).reshape(N_EDGES)
    ewp = jnp.concatenate([ew, jnp.zeros((pad,), jnp.float32)])
    ewp4 = ewp.reshape(TOT_SUP, SUPER, BLK)
    ewp = ewp.reshape(NW, NBLK, BLK)

    degp = _sc_deg(colp, ewp)
    degT = degp.T[:N_NODES]

    xw1, y1, dinv = _tc_l1(x, W1, degT)
    agg1 = _sc_agg(y1, rowp4, colp4, ewp4)
    xw2, y2 = _tc_l2(agg1[0], agg1[1], xw1, dinv, b1.reshape(1, D), W2)
    agg2 = _sc_agg(y2, rowp4, colp4, ewp4)
    return _tc_l3(agg2[0], agg2[1], xw2, dinv, b2.reshape(1, D), Wc,
                  bc.reshape(1, 2))
